# baseline (speedup 1.0000x reference)
import jax
import jax.numpy as jnp
from jax import lax
from jax.experimental import pallas as pl
from jax.experimental.pallas import tpu as pltpu
from jax.experimental.pallas import tpu_sc as plsc

VOCAB = 100000
EMB_DIM = 64
BATCH = 4096
SEQ = 50

NUM_WORKERS = 32
D_PER_W = EMB_DIM // NUM_WORKERS


def _emb_kernel(idx_hbm, tab_hbm, out_hbm,
                idx_sp, rowbuf, ib0, ib1, ob0, ob1,
                isem0, isem1, osem0, osem1):
    cid = lax.axis_index("c")
    sid = lax.axis_index("s")
    wid = sid * 2 + cid

    shard = SEQ * BATCH // 16
    for t in range(16):
        @pl.when(sid == t)
        def _():
            pltpu.sync_copy(
                idx_hbm.at[pl.ds(t * shard, shard)],
                idx_sp.at[pl.ds(t * shard, shard)],
            )

    plsc.subcore_barrier()

    ibufs = (ib0, ib1)
    obufs = (ob0, ob1)
    isems = (isem0, isem1)
    osems = (osem0, osem1)

    def iread(s):
        return pltpu.make_async_copy(
            idx_sp.at[pl.ds(s * BATCH, BATCH)], ibufs[s % 2], isems[s % 2]
        )

    def ocopy(s, d):
        return pltpu.make_async_copy(
            obufs[s % 2],
            out_hbm.at[pl.ds((s * EMB_DIM + d) * BATCH, BATCH)],
            osems[s % 2],
        )

    def compute(s):
        ib = ibufs[s % 2]
        ob = obufs[s % 2]

        @plsc.parallel_loop(0, BATCH, step=16, unroll=4)
        def _(k):
            idx16 = ib[pl.ds(k, 16)]
            ob[pl.ds(k, 16)] = plsc.load_gather(rowbuf, [idx16])

    for rep in range(D_PER_W):
        d = wid + NUM_WORKERS * rep
        pltpu.sync_copy(tab_hbm.at[pl.ds(d * VOCAB, VOCAB)], rowbuf)
        iread(0).start()
        for s in range(SEQ):
            iread(s).wait()
            if s + 1 < SEQ:
                iread(s + 1).start()
            if s >= 2 or rep > 0:
                ocopy(s % 2, 0).wait()
            compute(s)
            ocopy(s, d).start()
    ocopy(0, 0).wait()
    ocopy(1, 0).wait()


@jax.jit
def _embedding_lookup(idx_t, tab_t):
    mesh = plsc.VectorSubcoreMesh(core_axis_name="c", subcore_axis_name="s")
    f = pl.kernel(
        _emb_kernel,
        out_type=jax.ShapeDtypeStruct((SEQ * EMB_DIM * BATCH,), jnp.float32),
        mesh=mesh,
        scratch_types=[
            pltpu.MemorySpace.VMEM_SHARED((SEQ * BATCH,), jnp.int32),
            pltpu.VMEM((VOCAB,), jnp.float32),
            pltpu.VMEM((BATCH,), jnp.int32),
            pltpu.VMEM((BATCH,), jnp.int32),
            pltpu.VMEM((BATCH,), jnp.float32),
            pltpu.VMEM((BATCH,), jnp.float32),
            pltpu.SemaphoreType.DMA,
            pltpu.SemaphoreType.DMA,
            pltpu.SemaphoreType.DMA,
            pltpu.SemaphoreType.DMA,
        ],
        compiler_params=pltpu.CompilerParams(needs_layout_passes=False),
    )
    return f(idx_t, tab_t)


def kernel(inputs, emb_mat):
    idx_t = jnp.transpose(inputs, (1, 0)).astype(jnp.int32)
    tab_t = jnp.transpose(emb_mat, (1, 0))
    out_t = _embedding_lookup(
        idx_t.reshape(SEQ * BATCH), tab_t.reshape(EMB_DIM * VOCAB)
    ).reshape(SEQ, EMB_DIM, BATCH)
    return jnp.transpose(out_t, (2, 0, 1))


# speedup vs baseline: 2.0391x; 2.0391x over previous
import jax
import jax.numpy as jnp
from jax import lax
from jax.experimental import pallas as pl
from jax.experimental.pallas import tpu as pltpu
from jax.experimental.pallas import tpu_sc as plsc

VOCAB = 100000
EMB_DIM = 64
BATCH = 4096
SEQ = 50

NUM_WORKERS = 32
D_PER_W = EMB_DIM // NUM_WORKERS


def _emb_kernel(idx_hbm, tab_hbm, out_hbm,
                idx_sp, rowbuf, ib0, ib1, ob0, ob1,
                isem0, isem1, osem0, osem1):
    cid = lax.axis_index("c")
    sid = lax.axis_index("s")
    wid = sid * 2 + cid

    shard = SEQ * BATCH // 16
    for t in range(16):
        @pl.when(sid == t)
        def _():
            pltpu.sync_copy(
                idx_hbm.at[pl.ds(t * shard, shard)],
                idx_sp.at[pl.ds(t * shard, shard)],
            )

    plsc.subcore_barrier()

    ibufs = (ib0, ib1)
    obufs = (ob0, ob1)
    isems = (isem0, isem1)
    osems = (osem0, osem1)

    def iread(s):
        return pltpu.make_async_copy(
            idx_sp.at[pl.ds(s * BATCH, BATCH)], ibufs[s % 2], isems[s % 2]
        )

    def ocopy(s, d):
        return pltpu.make_async_copy(
            obufs[s % 2], out_hbm.at[s, d], osems[s % 2]
        )

    def compute(s):
        ib = ibufs[s % 2]
        ob = obufs[s % 2]

        @plsc.parallel_loop(0, BATCH, step=16, unroll=4)
        def _(k):
            idx16 = ib[pl.ds(k, 16)]
            ob[pl.ds(k, 16)] = plsc.load_gather(rowbuf, [idx16])

    for rep in range(D_PER_W):
        d = wid + NUM_WORKERS * rep
        pltpu.sync_copy(tab_hbm.at[d], rowbuf)
        iread(0).start()
        for s in range(SEQ):
            iread(s).wait()
            if s + 1 < SEQ:
                iread(s + 1).start()
            if s >= 2 or rep > 0:
                ocopy(s % 2, 0).wait()
            compute(s)
            ocopy(s, d).start()
    ocopy(0, 0).wait()
    ocopy(1, 0).wait()


@jax.jit
def _embedding_lookup(idx_t, tab_t):
    mesh = plsc.VectorSubcoreMesh(core_axis_name="c", subcore_axis_name="s")
    f = pl.kernel(
        _emb_kernel,
        out_type=jax.ShapeDtypeStruct((SEQ, EMB_DIM, BATCH), jnp.float32),
        mesh=mesh,
        scratch_types=[
            pltpu.MemorySpace.VMEM_SHARED((SEQ * BATCH,), jnp.int32),
            pltpu.VMEM((VOCAB,), jnp.float32),
            pltpu.VMEM((BATCH,), jnp.int32),
            pltpu.VMEM((BATCH,), jnp.int32),
            pltpu.VMEM((BATCH,), jnp.float32),
            pltpu.VMEM((BATCH,), jnp.float32),
            pltpu.SemaphoreType.DMA,
            pltpu.SemaphoreType.DMA,
            pltpu.SemaphoreType.DMA,
            pltpu.SemaphoreType.DMA,
        ],
        compiler_params=pltpu.CompilerParams(needs_layout_passes=False),
    )
    return f(idx_t, tab_t)


def kernel(inputs, emb_mat):
    idx_t = jnp.transpose(inputs, (1, 0)).astype(jnp.int32)
    tab_t = jnp.transpose(emb_mat, (1, 0))
    out_t = _embedding_lookup(idx_t.reshape(SEQ * BATCH), tab_t)
    return jnp.transpose(out_t, (2, 0, 1))


# speedup vs baseline: 2.0860x; 1.0230x over previous
import jax
import jax.numpy as jnp
from jax import lax
from jax.experimental import pallas as pl
from jax.experimental.pallas import tpu as pltpu
from jax.experimental.pallas import tpu_sc as plsc

VOCAB = 100000
EMB_DIM = 64
BATCH = 4096
SEQ = 50

NUM_WORKERS = 32
D_PER_W = EMB_DIM // NUM_WORKERS


def _emb_kernel(idx_hbm, tab_hbm, out_hbm,
                idx_sp, rowbuf, ib0, ib1, ob0, ob1,
                isem0, isem1, osem0, osem1):
    cid = lax.axis_index("c")
    sid = lax.axis_index("s")
    wid = sid * 2 + cid

    shard = SEQ * BATCH // 16
    for t in range(16):
        @pl.when(sid == t)
        def _():
            pltpu.sync_copy(
                idx_hbm.at[pl.ds(t * shard, shard)],
                idx_sp.at[pl.ds(t * shard, shard)],
            )

    plsc.subcore_barrier()

    ibufs = (ib0, ib1)
    obufs = (ob0, ob1)
    isems = (isem0, isem1)
    osems = (osem0, osem1)

    def iread(s):
        return pltpu.make_async_copy(
            idx_sp.at[pl.ds(s * BATCH, BATCH)], ibufs[s % 2], isems[s % 2]
        )

    def ocopy(s, d):
        return pltpu.make_async_copy(
            obufs[s % 2], out_hbm.at[s, d], osems[s % 2]
        )

    def compute(s):
        ib = ibufs[s % 2]
        ob = obufs[s % 2]

        @plsc.parallel_loop(0, BATCH, step=16, unroll=8)
        def _(k):
            idx16 = ib[pl.ds(k, 16)]
            ob[pl.ds(k, 16)] = plsc.load_gather(rowbuf, [idx16])

    for rep in range(D_PER_W):
        d = wid + NUM_WORKERS * rep
        pltpu.sync_copy(tab_hbm.at[d], rowbuf)
        iread(0).start()
        for s in range(SEQ):
            iread(s).wait()
            if s + 1 < SEQ:
                iread(s + 1).start()
            if s >= 2 or rep > 0:
                ocopy(s % 2, 0).wait()
            compute(s)
            ocopy(s, d).start()
    ocopy(0, 0).wait()
    ocopy(1, 0).wait()


@jax.jit
def _embedding_lookup(idx_t, tab_t):
    mesh = plsc.VectorSubcoreMesh(core_axis_name="c", subcore_axis_name="s")
    f = pl.kernel(
        _emb_kernel,
        out_type=jax.ShapeDtypeStruct((SEQ, EMB_DIM, BATCH), jnp.float32),
        mesh=mesh,
        scratch_types=[
            pltpu.MemorySpace.VMEM_SHARED((SEQ * BATCH,), jnp.int32),
            pltpu.VMEM((VOCAB,), jnp.float32),
            pltpu.VMEM((BATCH,), jnp.int32),
            pltpu.VMEM((BATCH,), jnp.int32),
            pltpu.VMEM((BATCH,), jnp.float32),
            pltpu.VMEM((BATCH,), jnp.float32),
            pltpu.SemaphoreType.DMA,
            pltpu.SemaphoreType.DMA,
            pltpu.SemaphoreType.DMA,
            pltpu.SemaphoreType.DMA,
        ],
        compiler_params=pltpu.CompilerParams(needs_layout_passes=False),
    )
    return f(idx_t, tab_t)


def kernel(inputs, emb_mat):
    idx_t = jnp.transpose(inputs, (1, 0)).astype(jnp.int32)
    tab_t = jnp.transpose(emb_mat, (1, 0))
    out_t = _embedding_lookup(idx_t.reshape(SEQ * BATCH), tab_t)
    return jnp.transpose(out_t, (2, 0, 1))


# speedup vs baseline: 2.1276x; 1.0200x over previous
import jax
import jax.numpy as jnp
from jax import lax
from jax.experimental import pallas as pl
from jax.experimental.pallas import tpu as pltpu
from jax.experimental.pallas import tpu_sc as plsc

VOCAB = 100000
EMB_DIM = 64
BATCH = 4096
SEQ = 50

NUM_WORKERS = 32
D_PER_W = EMB_DIM // NUM_WORKERS


def _emb_kernel(idx_hbm, tab_hbm, out_hbm,
                idx_sp, rowbuf, ib0, ib1, ob0, ob1,
                isem0, isem1, osem0, osem1, rsem):
    cid = lax.axis_index("c")
    sid = lax.axis_index("s")
    wid = sid * 2 + cid

    def rowcopy(d):
        return pltpu.make_async_copy(tab_hbm.at[d], rowbuf, rsem)

    rowcopy(wid).start()

    shard = SEQ * BATCH // 16
    for t in range(16):
        @pl.when(sid == t)
        def _():
            pltpu.sync_copy(
                idx_hbm.at[pl.ds(t * shard, shard)],
                idx_sp.at[pl.ds(t * shard, shard)],
            )

    plsc.subcore_barrier()

    ibufs = (ib0, ib1)
    obufs = (ob0, ob1)
    isems = (isem0, isem1)
    osems = (osem0, osem1)

    def iread(s):
        return pltpu.make_async_copy(
            idx_sp.at[pl.ds(s * BATCH, BATCH)], ibufs[s % 2], isems[s % 2]
        )

    def ocopy(s, d):
        return pltpu.make_async_copy(
            obufs[s % 2], out_hbm.at[s, d], osems[s % 2]
        )

    def compute(s):
        ib = ibufs[s % 2]
        ob = obufs[s % 2]

        @plsc.parallel_loop(0, BATCH, step=16, unroll=8)
        def _(k):
            idx16 = ib[pl.ds(k, 16)]
            ob[pl.ds(k, 16)] = plsc.load_gather(rowbuf, [idx16])

    for rep in range(D_PER_W):
        d = wid + NUM_WORKERS * rep
        if rep > 0:
            rowcopy(d).start()
        rowcopy(d).wait()
        iread(0).start()
        for s in range(SEQ):
            iread(s).wait()
            if s + 1 < SEQ:
                iread(s + 1).start()
            if s >= 2 or rep > 0:
                ocopy(s % 2, 0).wait()
            compute(s)
            ocopy(s, d).start()
    ocopy(0, 0).wait()
    ocopy(1, 0).wait()


@jax.jit
def _embedding_lookup(idx_t, tab_t):
    mesh = plsc.VectorSubcoreMesh(core_axis_name="c", subcore_axis_name="s")
    f = pl.kernel(
        _emb_kernel,
        out_type=jax.ShapeDtypeStruct((SEQ, EMB_DIM, BATCH), jnp.float32),
        mesh=mesh,
        scratch_types=[
            pltpu.MemorySpace.VMEM_SHARED((SEQ * BATCH,), jnp.int32),
            pltpu.VMEM((VOCAB,), jnp.float32),
            pltpu.VMEM((BATCH,), jnp.int32),
            pltpu.VMEM((BATCH,), jnp.int32),
            pltpu.VMEM((BATCH,), jnp.float32),
            pltpu.VMEM((BATCH,), jnp.float32),
            pltpu.SemaphoreType.DMA,
            pltpu.SemaphoreType.DMA,
            pltpu.SemaphoreType.DMA,
            pltpu.SemaphoreType.DMA,
            pltpu.SemaphoreType.DMA,
        ],
        compiler_params=pltpu.CompilerParams(needs_layout_passes=False),
    )
    return f(idx_t, tab_t)


def kernel(inputs, emb_mat):
    idx_t = jnp.transpose(inputs, (1, 0)).astype(jnp.int32)
    tab_t = jnp.transpose(emb_mat, (1, 0))
    out_t = _embedding_lookup(idx_t.reshape(SEQ * BATCH), tab_t)
    return jnp.transpose(out_t, (2, 0, 1))
